# Initial kernel scaffold; baseline (speedup 1.0000x reference)
import jax
import jax.numpy as jnp
from jax.experimental import pallas as pl


def kernel(xyz_sampled, c0, c1, c2, f0, f1, f2, u0, u1, u2, Wc, Wf):
    raise NotImplementedError("write your pallas kernel here")


# speedup vs baseline: 13.7994x; 13.7994x over previous
import functools

import jax
import jax.numpy as jnp
from jax import lax
from jax.experimental import pallas as pl
from jax.experimental.pallas import tpu as pltpu
from jax.experimental.pallas import tpu_sc as plsc

N = 262144
R = 64
F_DIM = 27
DC = 128
DF = 300

_INFO = plsc.get_sparse_core_info()
NC, NS, L = _INFO.num_cores, _INFO.num_subcores, _INFO.num_lanes
NW = NC * NS
PTS_PER_W = N // NW
CHUNK = 128
NCHUNKS = PTS_PER_W // CHUNK
GROUPS = CHUNK // L

_mesh = plsc.VectorSubcoreMesh(core_axis_name="c", subcore_axis_name="s")


@functools.partial(
    pl.kernel,
    out_type=(
        jax.ShapeDtypeStruct((R, N), jnp.float32),
        jax.ShapeDtypeStruct((R, N), jnp.float32),
        jax.ShapeDtypeStruct((N,), jnp.float32),
    ),
    mesh=_mesh,
    compiler_params=pltpu.CompilerParams(needs_layout_passes=False),
    scratch_types=[
        pltpu.VMEM((R * DC,), jnp.float32),
        pltpu.VMEM((R * DC,), jnp.float32),
        pltpu.VMEM((R * DC,), jnp.float32),
        pltpu.VMEM((R * DF,), jnp.float32),
        pltpu.VMEM((R * DF,), jnp.float32),
        pltpu.VMEM((R * DF,), jnp.float32),
        pltpu.VMEM((R * DC,), jnp.float32),
        pltpu.VMEM((R * DC,), jnp.float32),
        pltpu.VMEM((R * DC,), jnp.float32),
        pltpu.VMEM((CHUNK,), jnp.float32),
        pltpu.VMEM((CHUNK,), jnp.float32),
        pltpu.VMEM((CHUNK,), jnp.float32),
        pltpu.VMEM((R, CHUNK), jnp.float32),
        pltpu.VMEM((R, CHUNK), jnp.float32),
        pltpu.VMEM((CHUNK,), jnp.float32),
    ],
)
def _sc_features(x_hbm, y_hbm, z_hbm,
                 c0h, c1h, c2h, f0h, f1h, f2h, u0h, u1h, u2h,
                 fc_hbm, ff_hbm, un_hbm,
                 c0v, c1v, c2v, f0v, f1v, f2v, u0v, u1v, u2v,
                 xv, yv, zv, fcv, ffv, unv):
    wid = lax.axis_index("s") * NC + lax.axis_index("c")
    base = wid * PTS_PER_W

    pltpu.sync_copy(c0h, c0v)
    pltpu.sync_copy(c1h, c1v)
    pltpu.sync_copy(c2h, c2v)
    pltpu.sync_copy(f0h, f0v)
    pltpu.sync_copy(f1h, f1v)
    pltpu.sync_copy(f2h, f2v)
    pltpu.sync_copy(u0h, u0v)
    pltpu.sync_copy(u1h, u1v)
    pltpu.sync_copy(u2h, u2v)

    def idx_weights(t, d):
        pix = t * jnp.float32(d - 1)
        i0 = jnp.clip(pix.astype(jnp.int32), 0, d - 2)
        w1 = pix - i0.astype(jnp.float32)
        w0 = 1.0 - w1
        return i0, i0 + 1, w0, w1

    def chunk_body(ci, carry):
        off = base + ci * CHUNK
        pltpu.sync_copy(x_hbm.at[pl.ds(off, CHUNK)], xv)
        pltpu.sync_copy(y_hbm.at[pl.ds(off, CHUNK)], yv)
        pltpu.sync_copy(z_hbm.at[pl.ds(off, CHUNK)], zv)

        def group_body(g, carry2):
            s = g * L
            xx = xv[pl.ds(s, L)]
            yy = yv[pl.ds(s, L)]
            zz = zv[pl.ds(s, L)]
            ax0, ax1, awx0, awx1 = idx_weights(xx, DC)
            ay0, ay1, awy0, awy1 = idx_weights(yy, DC)
            az0, az1, awz0, awz1 = idx_weights(zz, DC)
            bx0, bx1, bwx0, bwx1 = idx_weights(xx, DF)
            by0, by1, bwy0, bwy1 = idx_weights(yy, DF)
            bz0, bz1, bwz0, bwz1 = idx_weights(zz, DF)

            def r_body(r, uacc):
                rc = jnp.full((L,), r * DC, jnp.int32)
                rf = jnp.full((L,), r * DF, jnp.int32)

                def interp(tab, rbase, i0, i1, w0, w1):
                    return (plsc.load_gather(tab, [rbase + i0]) * w0
                            + plsc.load_gather(tab, [rbase + i1]) * w1)

                fc = (interp(c0v, rc, ax0, ax1, awx0, awx1)
                      * interp(c1v, rc, ay0, ay1, awy0, awy1)
                      * interp(c2v, rc, az0, az1, awz0, awz1))
                fcv[r, pl.ds(s, L)] = fc
                ff = (interp(f0v, rf, bx0, bx1, bwx0, bwx1)
                      * interp(f1v, rf, by0, by1, bwy0, bwy1)
                      * interp(f2v, rf, bz0, bz1, bwz0, bwz1))
                ffv[r, pl.ds(s, L)] = ff
                uu = (interp(u0v, rc, ax0, ax1, awx0, awx1)
                      * interp(u1v, rc, ay0, ay1, awy0, awy1)
                      * interp(u2v, rc, az0, az1, awz0, awz1))
                return uacc + uu

            uacc = lax.fori_loop(0, R, r_body, jnp.zeros((L,), jnp.float32))
            unv[pl.ds(s, L)] = uacc
            return carry2

        lax.fori_loop(0, GROUPS, group_body, 0)
        pltpu.sync_copy(fcv, fc_hbm.at[:, pl.ds(off, CHUNK)])
        pltpu.sync_copy(ffv, ff_hbm.at[:, pl.ds(off, CHUNK)])
        pltpu.sync_copy(unv, un_hbm.at[pl.ds(off, CHUNK)])
        return carry

    lax.fori_loop(0, NCHUNKS, chunk_body, 0)


BN = 2048


def _tc_project_body(fc_ref, ff_ref, wc_ref, wf_ref, out_ref):
    oc = lax.dot_general(fc_ref[...], wc_ref[...], (((0,), (1,)), ((), ())),
                         preferred_element_type=jnp.float32)
    of = lax.dot_general(ff_ref[...], wf_ref[...], (((0,), (1,)), ((), ())),
                         preferred_element_type=jnp.float32)
    out_ref[...] = jnp.concatenate([oc, of], axis=1)


_tc_project = pl.pallas_call(
    _tc_project_body,
    grid=(N // BN,),
    in_specs=[
        pl.BlockSpec((R, BN), lambda i: (0, i)),
        pl.BlockSpec((R, BN), lambda i: (0, i)),
        pl.BlockSpec((F_DIM, R), lambda i: (0, 0)),
        pl.BlockSpec((F_DIM, R), lambda i: (0, 0)),
    ],
    out_specs=pl.BlockSpec((BN, 2 * F_DIM), lambda i: (i, 0)),
    out_shape=jax.ShapeDtypeStruct((N, 2 * F_DIM), jnp.float32),
)


@jax.jit
def kernel(xyz_sampled, c0, c1, c2, f0, f1, f2, u0, u1, u2, Wc, Wf):
    x = xyz_sampled[:, 0]
    y = xyz_sampled[:, 1]
    z = xyz_sampled[:, 2]
    fc, ff, un = _sc_features(
        x, y, z,
        c0.reshape(-1), c1.reshape(-1), c2.reshape(-1),
        f0.reshape(-1), f1.reshape(-1), f2.reshape(-1),
        u0.reshape(-1), u1.reshape(-1), u2.reshape(-1),
    )
    cat = _tc_project(fc, ff, Wc, Wf)
    return cat, un[:, None]
